# Initial kernel scaffold; baseline (speedup 1.0000x reference)
import jax
import jax.numpy as jnp
from jax.experimental import pallas as pl


def kernel(X_train, X_test, y_train):
    raise NotImplementedError("write your pallas kernel here")


# speedup vs baseline: 2.2677x; 2.2677x over previous
import functools

import jax
import jax.numpy as jnp
from jax import lax
from jax.experimental import pallas as pl
from jax.experimental.pallas import tpu as pltpu
from jax.experimental.pallas import tpu_sc as plsc

_KNN = 5
_BIG = 3.0e38
_IMAX = 2**31 - 1
_NC = 2
_NS = 16
_NW = _NC * _NS


def _make_topk(Q, D, K, TK, T):
    def body(x_ref, qt_ref, out_ref, run_d_ref, run_i_ref):
        t = pl.program_id(0)

        @pl.when(t == 0)
        def _init():
            run_d_ref[...] = jnp.full((8, Q), _BIG, jnp.float32)
            run_i_ref[...] = jnp.full((8, Q), _IMAX, jnp.int32)

        x = x_ref[...]
        qt = qt_ref[...]
        k_sq = jnp.sum(x * x, axis=1, keepdims=True)
        q_sq = jnp.sum(qt * qt, axis=0, keepdims=True)
        mm = jnp.dot(x, qt, preferred_element_type=jnp.float32)
        d2 = q_sq + k_sq - 2.0 * mm
        dist = jnp.sqrt(jnp.maximum(d2, 0.0))
        gidx = lax.broadcasted_iota(jnp.int32, (TK, Q), 0) + t * TK

        run_d = run_d_ref[...]
        run_i = run_i_ref[...]
        for j in range(_KNN):
            m = jnp.minimum(
                jnp.min(dist, axis=0, keepdims=True),
                jnp.min(run_d, axis=0, keepdims=True),
            )
            pw = jnp.min(jnp.where(dist == m, gidx, _IMAX), axis=0, keepdims=True)
            pr = jnp.min(jnp.where(run_d == m, run_i, _IMAX), axis=0, keepdims=True)
            pick = jnp.minimum(pw, pr)
            run_d_ref[j:j + 1, :] = m
            run_i_ref[j:j + 1, :] = pick
            dist = jnp.where(gidx == pick, _BIG, dist)
            run_d = jnp.where(run_i == pick, _BIG, run_d)

        @pl.when(t == T - 1)
        def _out():
            out_ref[...] = run_i_ref[...]

    return pl.pallas_call(
        body,
        grid=(T,),
        in_specs=[
            pl.BlockSpec((TK, D), lambda t: (t, 0)),
            pl.BlockSpec((D, Q), lambda t: (0, 0)),
        ],
        out_specs=pl.BlockSpec((8, Q), lambda t: (0, 0)),
        out_shape=jax.ShapeDtypeStruct((8, Q), jnp.int32),
        scratch_shapes=[
            pltpu.VMEM((8, Q), jnp.float32),
            pltpu.VMEM((8, Q), jnp.int32),
        ],
    )


def _make_vote(Q):
    qpw = Q // _NW
    assert qpw % 16 == 0 and (qpw * _KNN) % 8 == 0
    mesh = plsc.VectorSubcoreMesh(core_axis_name="c", subcore_axis_name="s")

    @functools.partial(
        pl.kernel,
        mesh=mesh,
        out_type=jax.ShapeDtypeStruct((Q,), jnp.int32),
        scratch_types=[
            pltpu.VMEM((_KNN, qpw), jnp.int32),
            pltpu.VMEM((_KNN, qpw), jnp.int32),
            pltpu.VMEM((qpw,), jnp.int32),
            pltpu.SemaphoreType.DMA,
        ],
    )
    def vote(idx_hbm, y_hbm, out_hbm, idx_v, lab_v, pred_v, sem):
        wid = lax.axis_index("s") * _NC + lax.axis_index("c")
        base = wid * qpw
        for j in range(_KNN):
            pltpu.sync_copy(idx_hbm.at[j, pl.ds(base, qpw)], idx_v.at[j])
        copies = [
            pltpu.async_copy(y_hbm.at[idx_v.at[j]], lab_v.at[j], sem)
            for j in range(_KNN)
        ]
        for c in copies:
            c.wait()
        one = jnp.full((16,), 1, jnp.int32)
        zero = jnp.zeros((16,), jnp.int32)
        for s in range(qpw // 16):
            sl = pl.ds(s * 16, 16)
            l = [lab_v[j, sl] for j in range(_KNN)]
            pred = None
            best = None
            for j in range(_KNN):
                cnt = zero
                for j2 in range(_KNN):
                    cnt = cnt + jnp.where(l[j] == l[j2], one, zero)
                score = cnt * 1000 - l[j]
                if j == 0:
                    pred, best = l[j], score
                else:
                    upd = score > best
                    pred = jnp.where(upd, l[j], pred)
                    best = jnp.where(upd, score, best)
            pred_v[sl] = pred
        pltpu.sync_copy(pred_v, out_hbm.at[pl.ds(base, qpw)])

    return vote


def kernel(X_train, X_test, y_train):
    K, D = X_train.shape
    Q = X_test.shape[0]
    TK = 1024
    T = -(-K // TK)
    Kp = T * TK
    xp = jnp.pad(X_train, ((0, Kp - K), (0, 0)), constant_values=1.0e15)
    qt = jnp.transpose(X_test)
    top = _make_topk(Q, D, K, TK, T)(xp, qt)
    idx5 = top[:_KNN]
    return _make_vote(Q)(idx5, y_train)


# speedup vs baseline: 2.4896x; 1.0979x over previous
import functools

import jax
import jax.numpy as jnp
from jax import lax
from jax.experimental import pallas as pl
from jax.experimental.pallas import tpu as pltpu
from jax.experimental.pallas import tpu_sc as plsc

_KNN = 5
_BIG = 3.0e38
_IMAX = 2**31 - 1
_NC = 2
_NS = 16
_NW = _NC * _NS


def _make_topk(Q, D, K, TK, T):
    def body(x_ref, qt_ref, out_ref, run_d_ref, run_i_ref):
        t = pl.program_id(0)

        @pl.when(t == 0)
        def _init():
            run_d_ref[...] = jnp.full((8, Q), _BIG, jnp.float32)
            run_i_ref[...] = jnp.full((8, Q), _IMAX, jnp.int32)

        x = x_ref[...]
        qt = qt_ref[...]
        k_sq = jnp.sum(x * x, axis=1, keepdims=True)
        q_sq = jnp.sum(qt * qt, axis=0, keepdims=True)
        mm = jnp.dot(x, qt, preferred_element_type=jnp.float32)
        d2 = q_sq + k_sq - 2.0 * mm
        dist = jnp.sqrt(jnp.maximum(d2, 0.0))
        gidx = lax.broadcasted_iota(jnp.int32, (TK, Q), 0) + t * TK

        run_d = run_d_ref[...]
        run_i = run_i_ref[...]
        for j in range(_KNN):
            m = jnp.minimum(
                jnp.min(dist, axis=0, keepdims=True),
                jnp.min(run_d, axis=0, keepdims=True),
            )
            pw = jnp.min(jnp.where(dist == m, gidx, _IMAX), axis=0, keepdims=True)
            pr = jnp.min(jnp.where(run_d == m, run_i, _IMAX), axis=0, keepdims=True)
            pick = jnp.minimum(pw, pr)
            run_d_ref[j:j + 1, :] = m
            run_i_ref[j:j + 1, :] = pick
            dist = jnp.where(gidx == pick, _BIG, dist)
            run_d = jnp.where(run_i == pick, _BIG, run_d)

        @pl.when(t == T - 1)
        def _out():
            out_ref[...] = run_i_ref[...]

    return pl.pallas_call(
        body,
        grid=(T,),
        in_specs=[
            pl.BlockSpec((TK, D), lambda t: (t, 0)),
            pl.BlockSpec((D, Q), lambda t: (0, 0)),
        ],
        out_specs=pl.BlockSpec((8, Q), lambda t: (0, 0)),
        out_shape=jax.ShapeDtypeStruct((8, Q), jnp.int32),
        scratch_shapes=[
            pltpu.VMEM((8, Q), jnp.float32),
            pltpu.VMEM((8, Q), jnp.int32),
        ],
    )


def _make_vote(Q):
    qpw = Q // _NW
    assert qpw % 16 == 0 and (qpw * _KNN) % 8 == 0
    mesh = plsc.VectorSubcoreMesh(core_axis_name="c", subcore_axis_name="s")

    @functools.partial(
        pl.kernel,
        mesh=mesh,
        out_type=jax.ShapeDtypeStruct((Q,), jnp.int32),
        scratch_types=[
            pltpu.VMEM((_KNN, qpw), jnp.int32),
            pltpu.VMEM((_KNN, qpw), jnp.int32),
            pltpu.VMEM((qpw,), jnp.int32),
            pltpu.SemaphoreType.DMA,
        ],
    )
    def vote(idx_hbm, y_hbm, out_hbm, idx_v, lab_v, pred_v, sem):
        wid = lax.axis_index("s") * _NC + lax.axis_index("c")
        base = wid * qpw
        for j in range(_KNN):
            pltpu.sync_copy(idx_hbm.at[j, pl.ds(base, qpw)], idx_v.at[j])
        copies = [
            pltpu.async_copy(y_hbm.at[idx_v.at[j]], lab_v.at[j], sem)
            for j in range(_KNN)
        ]
        for c in copies:
            c.wait()
        one = jnp.full((16,), 1, jnp.int32)
        zero = jnp.zeros((16,), jnp.int32)
        for s in range(qpw // 16):
            sl = pl.ds(s * 16, 16)
            l = [lab_v[j, sl] for j in range(_KNN)]
            pred = None
            best = None
            for j in range(_KNN):
                cnt = zero
                for j2 in range(_KNN):
                    cnt = cnt + jnp.where(l[j] == l[j2], one, zero)
                score = cnt * 1000 - l[j]
                if j == 0:
                    pred, best = l[j], score
                else:
                    upd = score > best
                    pred = jnp.where(upd, l[j], pred)
                    best = jnp.where(upd, score, best)
            pred_v[sl] = pred
        pltpu.sync_copy(pred_v, out_hbm.at[pl.ds(base, qpw)])

    return vote


def kernel(X_train, X_test, y_train):
    K, D = X_train.shape
    Q = X_test.shape[0]
    TK = 2048
    T = -(-K // TK)
    Kp = T * TK
    xp = jnp.pad(X_train, ((0, Kp - K), (0, 0)), constant_values=1.0e15)
    qt = jnp.transpose(X_test)
    top = _make_topk(Q, D, K, TK, T)(xp, qt)
    idx5 = top[:_KNN]
    return _make_vote(Q)(idx5, y_train)


# speedup vs baseline: 2.5613x; 1.0288x over previous
import functools

import jax
import jax.numpy as jnp
from jax import lax
from jax.experimental import pallas as pl
from jax.experimental.pallas import tpu as pltpu
from jax.experimental.pallas import tpu_sc as plsc

_KNN = 5
_BIG = 3.0e38
_IMAX = 2**31 - 1
_NC = 2
_NS = 16
_NW = _NC * _NS


def _make_topk(Q, D, K, TK, T):
    def body(x_ref, qt_ref, out_ref, run_d_ref, run_i_ref):
        t = pl.program_id(0)

        @pl.when(t == 0)
        def _init():
            run_d_ref[...] = jnp.full((8, Q), _BIG, jnp.float32)
            run_i_ref[...] = jnp.full((8, Q), _IMAX, jnp.int32)

        x = x_ref[...]
        qt = qt_ref[...]
        k_sq = jnp.sum(x * x, axis=1, keepdims=True)
        q_sq = jnp.sum(qt * qt, axis=0, keepdims=True)
        mm = jnp.dot(x, qt, preferred_element_type=jnp.float32)
        d2 = q_sq + k_sq - 2.0 * mm
        dist = jnp.sqrt(jnp.maximum(d2, 0.0))
        gidx = lax.broadcasted_iota(jnp.int32, (TK, Q), 0) + t * TK

        run_d = run_d_ref[...]
        run_i = run_i_ref[...]
        for j in range(_KNN):
            m = jnp.minimum(
                jnp.min(dist, axis=0, keepdims=True),
                jnp.min(run_d, axis=0, keepdims=True),
            )
            pw = jnp.min(jnp.where(dist == m, gidx, _IMAX), axis=0, keepdims=True)
            pr = jnp.min(jnp.where(run_d == m, run_i, _IMAX), axis=0, keepdims=True)
            pick = jnp.minimum(pw, pr)
            run_d_ref[j:j + 1, :] = m
            run_i_ref[j:j + 1, :] = pick
            dist = jnp.where(gidx == pick, _BIG, dist)
            run_d = jnp.where(run_i == pick, _BIG, run_d)

        @pl.when(t == T - 1)
        def _out():
            out_ref[...] = run_i_ref[...]

    return pl.pallas_call(
        body,
        grid=(T,),
        in_specs=[
            pl.BlockSpec((TK, D), lambda t: (t, 0)),
            pl.BlockSpec((D, Q), lambda t: (0, 0)),
        ],
        out_specs=pl.BlockSpec((8, Q), lambda t: (0, 0)),
        out_shape=jax.ShapeDtypeStruct((8, Q), jnp.int32),
        scratch_shapes=[
            pltpu.VMEM((8, Q), jnp.float32),
            pltpu.VMEM((8, Q), jnp.int32),
        ],
    )


def _make_vote(Q):
    qpw = Q // _NW
    assert qpw % 16 == 0 and (qpw * _KNN) % 8 == 0
    mesh = plsc.VectorSubcoreMesh(core_axis_name="c", subcore_axis_name="s")

    @functools.partial(
        pl.kernel,
        mesh=mesh,
        out_type=jax.ShapeDtypeStruct((Q,), jnp.int32),
        scratch_types=[
            pltpu.VMEM((_KNN, qpw), jnp.int32),
            pltpu.VMEM((_KNN, qpw), jnp.int32),
            pltpu.VMEM((qpw,), jnp.int32),
            pltpu.SemaphoreType.DMA,
        ],
    )
    def vote(idx_hbm, y_hbm, out_hbm, idx_v, lab_v, pred_v, sem):
        wid = lax.axis_index("s") * _NC + lax.axis_index("c")
        base = wid * qpw
        for j in range(_KNN):
            pltpu.sync_copy(idx_hbm.at[j, pl.ds(base, qpw)], idx_v.at[j])
        copies = [
            pltpu.async_copy(y_hbm.at[idx_v.at[j]], lab_v.at[j], sem)
            for j in range(_KNN)
        ]
        for c in copies:
            c.wait()
        one = jnp.full((16,), 1, jnp.int32)
        zero = jnp.zeros((16,), jnp.int32)
        for s in range(qpw // 16):
            sl = pl.ds(s * 16, 16)
            l = [lab_v[j, sl] for j in range(_KNN)]
            pred = None
            best = None
            for j in range(_KNN):
                cnt = zero
                for j2 in range(_KNN):
                    cnt = cnt + jnp.where(l[j] == l[j2], one, zero)
                score = cnt * 1000 - l[j]
                if j == 0:
                    pred, best = l[j], score
                else:
                    upd = score > best
                    pred = jnp.where(upd, l[j], pred)
                    best = jnp.where(upd, score, best)
            pred_v[sl] = pred
        pltpu.sync_copy(pred_v, out_hbm.at[pl.ds(base, qpw)])

    return vote


def kernel(X_train, X_test, y_train):
    K, D = X_train.shape
    Q = X_test.shape[0]
    TK = 2000 if K % 2000 == 0 else 2048
    T = -(-K // TK)
    Kp = T * TK
    if Kp != K:
        X_train = jnp.pad(X_train, ((0, Kp - K), (0, 0)),
                          constant_values=1.0e15)
    qt = jnp.transpose(X_test)
    top = _make_topk(Q, D, K, TK, T)(X_train, qt)
    return _make_vote(Q)(top, y_train)


# speedup vs baseline: 2.5874x; 1.0102x over previous
import functools

import jax
import jax.numpy as jnp
from jax import lax
from jax.experimental import pallas as pl
from jax.experimental.pallas import tpu as pltpu
from jax.experimental.pallas import tpu_sc as plsc

_KNN = 5
_BIG = 3.0e38
_IMAX = 2**31 - 1
_NC = 2
_NS = 16
_NW = _NC * _NS


def _make_topk(Q, D, K, TK, T):
    def body(x_ref, qt_ref, out_ref, run_d_ref, run_i_ref):
        t = pl.program_id(0)

        @pl.when(t == 0)
        def _init():
            run_d_ref[...] = jnp.full((8, Q), _BIG, jnp.float32)
            run_i_ref[...] = jnp.full((8, Q), _IMAX, jnp.int32)

        x = x_ref[...]
        qt = qt_ref[...]
        k_sq = jnp.sum(x * x, axis=1, keepdims=True)
        q_sq = jnp.sum(qt * qt, axis=0, keepdims=True)
        mm = jnp.dot(x, qt, preferred_element_type=jnp.float32)
        d2 = q_sq + k_sq - 2.0 * mm
        dist = jnp.sqrt(jnp.maximum(d2, 0.0))
        gidx = lax.broadcasted_iota(jnp.int32, (TK, Q), 0) + t * TK

        run_d = run_d_ref[...]
        run_i = run_i_ref[...]
        for j in range(_KNN):
            m = jnp.minimum(
                jnp.min(dist, axis=0, keepdims=True),
                jnp.min(run_d, axis=0, keepdims=True),
            )
            off = lax.bitcast_convert_type(dist - m, jnp.int32)
            pw = jnp.min(gidx + off, axis=0, keepdims=True)
            pr = jnp.min(jnp.where(run_d == m, run_i, _IMAX), axis=0, keepdims=True)
            pick = jnp.minimum(pw, pr)
            run_d_ref[j:j + 1, :] = m
            run_i_ref[j:j + 1, :] = pick
            dist = jnp.where(gidx == pick, _BIG, dist)
            run_d = jnp.where(run_i == pick, _BIG, run_d)

        @pl.when(t == T - 1)
        def _out():
            out_ref[...] = run_i_ref[...]

    return pl.pallas_call(
        body,
        grid=(T,),
        in_specs=[
            pl.BlockSpec((TK, D), lambda t: (t, 0)),
            pl.BlockSpec((D, Q), lambda t: (0, 0)),
        ],
        out_specs=pl.BlockSpec((8, Q), lambda t: (0, 0)),
        out_shape=jax.ShapeDtypeStruct((8, Q), jnp.int32),
        scratch_shapes=[
            pltpu.VMEM((8, Q), jnp.float32),
            pltpu.VMEM((8, Q), jnp.int32),
        ],
    )


def _make_vote(Q):
    qpw = Q // _NW
    assert qpw % 16 == 0 and (qpw * _KNN) % 8 == 0
    mesh = plsc.VectorSubcoreMesh(core_axis_name="c", subcore_axis_name="s")

    @functools.partial(
        pl.kernel,
        mesh=mesh,
        out_type=jax.ShapeDtypeStruct((Q,), jnp.int32),
        scratch_types=[
            pltpu.VMEM((_KNN, qpw), jnp.int32),
            pltpu.VMEM((_KNN, qpw), jnp.int32),
            pltpu.VMEM((qpw,), jnp.int32),
            pltpu.SemaphoreType.DMA,
        ],
    )
    def vote(idx_hbm, y_hbm, out_hbm, idx_v, lab_v, pred_v, sem):
        wid = lax.axis_index("s") * _NC + lax.axis_index("c")
        base = wid * qpw
        for j in range(_KNN):
            pltpu.sync_copy(idx_hbm.at[j, pl.ds(base, qpw)], idx_v.at[j])
        copies = [
            pltpu.async_copy(y_hbm.at[idx_v.at[j]], lab_v.at[j], sem)
            for j in range(_KNN)
        ]
        for c in copies:
            c.wait()
        one = jnp.full((16,), 1, jnp.int32)
        zero = jnp.zeros((16,), jnp.int32)
        for s in range(qpw // 16):
            sl = pl.ds(s * 16, 16)
            l = [lab_v[j, sl] for j in range(_KNN)]
            pred = None
            best = None
            for j in range(_KNN):
                cnt = zero
                for j2 in range(_KNN):
                    cnt = cnt + jnp.where(l[j] == l[j2], one, zero)
                score = cnt * 1000 - l[j]
                if j == 0:
                    pred, best = l[j], score
                else:
                    upd = score > best
                    pred = jnp.where(upd, l[j], pred)
                    best = jnp.where(upd, score, best)
            pred_v[sl] = pred
        pltpu.sync_copy(pred_v, out_hbm.at[pl.ds(base, qpw)])

    return vote


def kernel(X_train, X_test, y_train):
    K, D = X_train.shape
    Q = X_test.shape[0]
    TK = 2000 if K % 2000 == 0 else 2048
    T = -(-K // TK)
    Kp = T * TK
    if Kp != K:
        X_train = jnp.pad(X_train, ((0, Kp - K), (0, 0)),
                          constant_values=1.0e15)
    qt = jnp.transpose(X_test)
    top = _make_topk(Q, D, K, TK, T)(X_train, qt)
    return _make_vote(Q)(top, y_train)
